# Initial kernel scaffold; baseline (speedup 1.0000x reference)
import jax
import jax.numpy as jnp
from jax.experimental import pallas as pl


def kernel(x, edge_index, W1_l, b1_l, W1_r, W2_l, b2_l, W2_r):
    raise NotImplementedError("write your pallas kernel here")


# speedup vs baseline: 12.7184x; 12.7184x over previous
import functools

import jax
import jax.numpy as jnp
from jax import lax
from jax.experimental import pallas as pl
from jax.experimental.pallas import tpu as pltpu
from jax.experimental.pallas import tpu_sc as plsc

N = 10000
E = 320000
D = 128

NC = 2
NS = 16
NW = NC * NS

B = 100
CHUNKS = 100
ROWS_PER_TILE = N // NS


def _zero_fill(ref, nrows, ncols):
  @pl.loop(0, nrows)
  def _(r):
    for k in range(ncols // 16):
      ref[r, pl.ds(16 * k, 16)] = jnp.zeros((16,), jnp.float32)


_MESH = plsc.VectorSubcoreMesh(core_axis_name="c", subcore_axis_name="s")
_SC_PARAMS = pltpu.CompilerParams(use_tc_tiling_on_sc=False)


def _sc_agg_body(feats, edges, acc_out,
                 src_idx, dst_idx, rows0, rows1, sem0, sem1, acc_sh):
  cid = lax.axis_index("c")
  sid = lax.axis_index("s")
  blk = cid * NS + sid
  base = sid * ROWS_PER_TILE

  _zero_fill(rows0, B, D)
  nfull = ROWS_PER_TILE // B
  tail = ROWS_PER_TILE - nfull * B
  for k in range(nfull):
    pltpu.sync_copy(rows0, acc_sh.at[pl.ds(base + k * B, B)])
  if tail:
    pltpu.sync_copy(rows0.at[pl.ds(0, tail)],
                    acc_sh.at[pl.ds(base + nfull * B, tail)])

  pltpu.sync_copy(edges.at[0, blk], src_idx)
  pltpu.sync_copy(edges.at[1, blk], dst_idx)

  plsc.subcore_barrier()

  def start(j, buf, sem):
    pltpu.async_copy(feats.at[src_idx.at[j]], buf, sem)

  def wait(buf, sem):
    pltpu.make_async_copy(feats.at[pl.ds(0, B)], buf, sem).wait()

  def scat(j, buf):
    pltpu.sync_copy(buf, acc_sh.at[dst_idx.at[j]], add=True)

  start(0, rows0, sem0)
  start(1, rows1, sem1)

  @pl.loop(0, CHUNKS // 2 - 1)
  def _(i):
    j0 = 2 * i
    wait(rows0, sem0)
    scat(j0, rows0)
    start(j0 + 2, rows0, sem0)
    wait(rows1, sem1)
    scat(j0 + 1, rows1)
    start(j0 + 3, rows1, sem1)

  wait(rows0, sem0)
  scat(CHUNKS - 2, rows0)
  wait(rows1, sem1)
  scat(CHUNKS - 1, rows1)

  plsc.subcore_barrier()

  pltpu.sync_copy(acc_sh.at[pl.ds(base, ROWS_PER_TILE)],
                  acc_out.at[cid, pl.ds(base, ROWS_PER_TILE)])


_sc_agg = pl.kernel(
    _sc_agg_body,
    out_type=[jax.ShapeDtypeStruct((NC, N, D), jnp.float32)],
    mesh=_MESH,
    scratch_types=[
        pltpu.VMEM((CHUNKS, B), jnp.int32),
        pltpu.VMEM((CHUNKS, B), jnp.int32),
        pltpu.VMEM((B, D), jnp.float32),
        pltpu.VMEM((B, D), jnp.float32),
        pltpu.SemaphoreType.DMA,
        pltpu.SemaphoreType.DMA,
        pltpu.VMEM_SHARED((N, D), jnp.float32),
    ],
    compiler_params=_SC_PARAMS)


def _sc_cnt_body(edges, cnt_out, dst_idx, ones_v, cnt_sh):
  cid = lax.axis_index("c")
  sid = lax.axis_index("s")
  blk = cid * NS + sid
  base = sid * ROWS_PER_TILE

  _zero_fill(ones_v, B, 16)
  nfull = ROWS_PER_TILE // B
  tail = ROWS_PER_TILE - nfull * B
  for k in range(nfull):
    pltpu.sync_copy(ones_v, cnt_sh.at[pl.ds(base + k * B, B)])
  if tail:
    pltpu.sync_copy(ones_v.at[pl.ds(0, tail)],
                    cnt_sh.at[pl.ds(base + nfull * B, tail)])

  @pl.loop(0, B)
  def _(r):
    ones_v[r, :] = jnp.ones((16,), jnp.float32)

  pltpu.sync_copy(edges.at[1, blk], dst_idx)

  plsc.subcore_barrier()

  @pl.loop(0, CHUNKS)
  def _(j):
    pltpu.sync_copy(ones_v, cnt_sh.at[dst_idx.at[j]], add=True)

  plsc.subcore_barrier()

  pltpu.sync_copy(cnt_sh.at[pl.ds(base, ROWS_PER_TILE)],
                  cnt_out.at[cid, pl.ds(base, ROWS_PER_TILE)])


_sc_cnt = pl.kernel(
    _sc_cnt_body,
    out_type=[jax.ShapeDtypeStruct((NC, N, 16), jnp.float32)],
    mesh=_MESH,
    scratch_types=[
        pltpu.VMEM((CHUNKS, B), jnp.int32),
        pltpu.VMEM((B, 16), jnp.float32),
        pltpu.VMEM_SHARED((N, 16), jnp.float32),
    ],
    compiler_params=_SC_PARAMS)


_RB = 1000
_GRID = N // _RB


def _dot(a, b):
  return lax.dot_general(a, b, (((1,), (0,)), ((), ())),
                         precision=lax.Precision.HIGHEST,
                         preferred_element_type=jnp.float32)


def _mm2_body(x_ref, wl_ref, wr_ref, ol_ref, or_ref):
  xb = x_ref[...]
  ol_ref[...] = _dot(xb, wl_ref[...])
  or_ref[...] = _dot(xb, wr_ref[...])


@jax.jit
def _mm2(x, wl, wr):
  return pl.pallas_call(
      _mm2_body,
      grid=(_GRID,),
      in_specs=[
          pl.BlockSpec((_RB, D), lambda i: (i, 0)),
          pl.BlockSpec((D, D), lambda i: (0, 0)),
          pl.BlockSpec((D, D), lambda i: (0, 0)),
      ],
      out_specs=[
          pl.BlockSpec((_RB, D), lambda i: (i, 0)),
          pl.BlockSpec((_RB, D), lambda i: (i, 0)),
      ],
      out_shape=[
          jax.ShapeDtypeStruct((N, D), jnp.float32),
          jax.ShapeDtypeStruct((N, D), jnp.float32),
      ],
  )(x, wl, wr)


def _mid_body(acc_ref, cnt_ref, xr_ref, b1_ref, wl_ref, wr_ref,
              ol_ref, or_ref):
  s = acc_ref[0] + acc_ref[1]
  c = cnt_ref[0, :, 0] + cnt_ref[1, :, 0]
  rc = 1.0 / jnp.maximum(c, 1.0)
  h = jnp.maximum(s * rc[:, None] + b1_ref[...] + xr_ref[...], 0.0)
  ol_ref[...] = _dot(h, wl_ref[...])
  or_ref[...] = _dot(h, wr_ref[...])


@jax.jit
def _mid(acc, cnt, xr, b1, wl, wr):
  return pl.pallas_call(
      _mid_body,
      grid=(_GRID,),
      in_specs=[
          pl.BlockSpec((NC, _RB, D), lambda i: (0, i, 0)),
          pl.BlockSpec((NC, _RB, 16), lambda i: (0, i, 0)),
          pl.BlockSpec((_RB, D), lambda i: (i, 0)),
          pl.BlockSpec((1, D), lambda i: (0, 0)),
          pl.BlockSpec((D, D), lambda i: (0, 0)),
          pl.BlockSpec((D, D), lambda i: (0, 0)),
      ],
      out_specs=[
          pl.BlockSpec((_RB, D), lambda i: (i, 0)),
          pl.BlockSpec((_RB, D), lambda i: (i, 0)),
      ],
      out_shape=[
          jax.ShapeDtypeStruct((N, D), jnp.float32),
          jax.ShapeDtypeStruct((N, D), jnp.float32),
      ],
  )(acc, cnt, xr, b1, wl, wr)


def _final_body(acc_ref, cnt_ref, hr_ref, b2_ref, o_ref):
  s = acc_ref[0] + acc_ref[1]
  c = cnt_ref[0, :, 0] + cnt_ref[1, :, 0]
  rc = 1.0 / jnp.maximum(c, 1.0)
  o_ref[...] = s * rc[:, None] + b2_ref[...] + hr_ref[...]


@jax.jit
def _final(acc, cnt, hr, b2):
  return pl.pallas_call(
      _final_body,
      grid=(_GRID,),
      in_specs=[
          pl.BlockSpec((NC, _RB, D), lambda i: (0, i, 0)),
          pl.BlockSpec((NC, _RB, 16), lambda i: (0, i, 0)),
          pl.BlockSpec((_RB, D), lambda i: (i, 0)),
          pl.BlockSpec((1, D), lambda i: (0, 0)),
      ],
      out_specs=pl.BlockSpec((_RB, D), lambda i: (i, 0)),
      out_shape=jax.ShapeDtypeStruct((N, D), jnp.float32),
  )(acc, cnt, hr, b2)


@jax.jit
def kernel(x, edge_index, W1_l, b1_l, W1_r, W2_l, b2_l, W2_r):
  edges = edge_index.reshape(2, NW, CHUNKS, B)
  xl, xr = _mm2(x, W1_l, W1_r)
  cnt, = _sc_cnt(edges)
  acc1, = _sc_agg(xl, edges)
  h2l, h2r = _mid(acc1, cnt, xr, b1_l.reshape(1, D), W2_l, W2_r)
  acc2, = _sc_agg(h2l, edges)
  return _final(acc2, cnt, h2r, b2_l.reshape(1, D))
